# Initial kernel scaffold; baseline (speedup 1.0000x reference)
import math

import jax
import jax.numpy as jnp
from jax import lax
from jax.experimental import pallas as pl
from jax.experimental.pallas import tpu as pltpu
from jax.experimental.pallas import tpu_sc as plsc

N = 10000
E = 320000
D = 128
H1 = 4
C = 6
B = 64

NC = 2
NS = 16
NW = NC * NS
LANES = 16

NPAD = 10016
ROWS_PT = NPAD // NS
CHUNK = 128
KC = -(-E // (NW * CHUNK))
EPAD = NW * KC * CHUNK

_ISQRT_C = 1.0 / math.sqrt(float(C))


def _make_edge_kernel(heads, ch, qw_, kw_, aw_):
    hc = heads * ch
    qe_off = hc
    v_off = kw_ // 2
    w_off = hc
    u_off = hc + heads
    mesh = plsc.VectorSubcoreMesh(core_axis_name="c", subcore_axis_name="s")

    def body(qt, kvt, src3, dst3, ea3, zr, out,
             idx_s, idx_d, ea_v, q_rows, kv_rows, out_rows, sem1, sem2):
        cid = lax.axis_index("c")
        sid = lax.axis_index("s")
        wid = cid * NS + sid

        def run(acc):
            pltpu.sync_copy(zr.at[pl.ds(sid * ROWS_PT, ROWS_PT)],
                            acc.at[pl.ds(sid * ROWS_PT, ROWS_PT)])
            plsc.subcore_barrier()

            def chunk_body(j, carry):
                pltpu.sync_copy(src3.at[wid, j], idx_s)
                pltpu.sync_copy(dst3.at[wid, j], idx_d)
                pltpu.sync_copy(ea3.at[wid, j], ea_v)
                cp1 = pltpu.async_copy(kvt.at[idx_s], kv_rows, sem1)
                cp2 = pltpu.async_copy(qt.at[idx_d], q_rows, sem2)
                cp1.wait()
                cp2.wait()
                for g in range(CHUNK // LANES):
                    lanes = lax.iota(jnp.int32, 16) + (g * LANES)
                    ea_g = ea_v[pl.ds(g * LANES, LANES)]
                    for h in range(heads):
                        qe = plsc.load_gather(
                            q_rows, [lanes, jnp.full((16,), qe_off + h, jnp.int32)])
                        alpha = ea_g * qe
                        for c in range(ch):
                            col = jnp.full((16,), h * ch + c, jnp.int32)
                            qv = plsc.load_gather(q_rows, [lanes, col])
                            kv = plsc.load_gather(kv_rows, [lanes, col])
                            alpha = alpha + qv * kv
                        w = jnp.exp(alpha)
                        plsc.store_scatter(
                            out_rows, [lanes, jnp.full((16,), w_off + h, jnp.int32)], w)
                        plsc.store_scatter(
                            out_rows, [lanes, jnp.full((16,), u_off + h, jnp.int32)],
                            w * ea_g)
                        for c in range(ch):
                            vv = plsc.load_gather(
                                kv_rows,
                                [lanes, jnp.full((16,), v_off + h * ch + c, jnp.int32)])
                            plsc.store_scatter(
                                out_rows,
                                [lanes, jnp.full((16,), h * ch + c, jnp.int32)],
                                w * vv)
                pltpu.sync_copy(out_rows, acc.at[idx_d], add=True)
                return carry

            lax.fori_loop(0, KC, chunk_body, 0)
            plsc.subcore_barrier()
            pltpu.sync_copy(acc.at[pl.ds(sid * ROWS_PT, ROWS_PT)],
                            out.at[cid, pl.ds(sid * ROWS_PT, ROWS_PT)])

        pl.run_scoped(run, pltpu.VMEM_SHARED((NPAD, aw_), jnp.float32))

    return pl.kernel(
        body,
        out_type=jax.ShapeDtypeStruct((NC, NPAD, aw_), jnp.float32),
        mesh=mesh,
        scratch_types=[
            pltpu.VMEM((CHUNK,), jnp.int32),
            pltpu.VMEM((CHUNK,), jnp.int32),
            pltpu.VMEM((CHUNK,), jnp.float32),
            pltpu.VMEM((CHUNK, qw_), jnp.float32),
            pltpu.VMEM((CHUNK, kw_), jnp.float32),
            pltpu.VMEM((CHUNK, aw_), jnp.float32),
            pltpu.SemaphoreType.DMA,
            pltpu.SemaphoreType.DMA,
        ],
    )


_edge_l1 = _make_edge_kernel(H1, C, 32, 64, 32)
_edge_l2 = _make_edge_kernel(1, C, 16, 16, 8)


def _elu(x):
    return jnp.where(x > 0, x, jnp.exp(jnp.minimum(x, 0.0)) - 1.0)


def _layernorm(x, w, b):
    mu = jnp.mean(x, axis=-1, keepdims=True)
    var = jnp.mean((x - mu) * (x - mu), axis=-1, keepdims=True)
    return (x - mu) * lax.rsqrt(var + 1e-5) * w + b


def _prep1_body(x_ref, qw_ref, qb_ref, kw_ref, kb_ref, vw_ref, vb_ref,
                e1w_ref, skw_ref, skb_ref, qt_ref, kvt_ref, sk_ref):
    x = x_ref[...]
    q = (jnp.dot(x, qw_ref[...], preferred_element_type=jnp.float32)
         + qb_ref[...]) * _ISQRT_C
    k = jnp.dot(x, kw_ref[...], preferred_element_type=jnp.float32) + kb_ref[...]
    v = jnp.dot(x, vw_ref[...], preferred_element_type=jnp.float32) + vb_ref[...]
    e1 = e1w_ref[...]
    qe_cols = []
    for h in range(H1):
        s = q[:, h * C:(h + 1) * C] * e1[:, h * C:(h + 1) * C]
        qe_cols.append(jnp.sum(s, axis=1, keepdims=True))
    z4 = jnp.zeros((NPAD, 4), jnp.float32)
    z8 = jnp.zeros((NPAD, 8), jnp.float32)
    qt_ref[...] = jnp.concatenate([q] + qe_cols + [z4], axis=1)
    kvt_ref[...] = jnp.concatenate([k, z8, v, z8], axis=1)
    sk_ref[...] = (jnp.dot(x, skw_ref[...], preferred_element_type=jnp.float32)
                   + skb_ref[...])


_prep1 = pl.pallas_call(
    _prep1_body,
    out_shape=(
        jax.ShapeDtypeStruct((NPAD, 32), jnp.float32),
        jax.ShapeDtypeStruct((NPAD, 64), jnp.float32),
        jax.ShapeDtypeStruct((NPAD, H1 * C), jnp.float32),
    ),
)


def _combine1_body(p_ref, sk_ref, e1w_ref, ln1w_ref, ln1b_ref,
                   q2w_ref, q2b_ref, k2w_ref, k2b_ref, v2w_ref, v2b_ref,
                   e2w_ref, sk2w_ref, sk2b_ref,
                   q2t_ref, kv2t_ref, sk2_ref):
    p = p_ref[0, :, :] + p_ref[1, :, :]
    e1 = e1w_ref[...]
    cols = []
    for h in range(H1):
        w = p[:, 24 + h:25 + h]
        u = p[:, 28 + h:29 + h]
        for c in range(C):
            j = h * C + c
            cols.append((p[:, j:j + 1] + u * e1[:, j:j + 1]) / (w + 1e-16))
    conv = jnp.concatenate(cols, axis=1) + sk_ref[...]
    h1 = _layernorm(_elu(conv), ln1w_ref[...], ln1b_ref[...])

    q2 = (jnp.dot(h1, q2w_ref[...], preferred_element_type=jnp.float32)
          + q2b_ref[...]) * _ISQRT_C
    k2 = jnp.dot(h1, k2w_ref[...], preferred_element_type=jnp.float32) + k2b_ref[...]
    v2 = jnp.dot(h1, v2w_ref[...], preferred_element_type=jnp.float32) + v2b_ref[...]
    qe2 = jnp.sum(q2 * e2w_ref[...], axis=1, keepdims=True)
    z9 = jnp.zeros((NPAD, 9), jnp.float32)
    z2 = jnp.zeros((NPAD, 2), jnp.float32)
    q2t_ref[...] = jnp.concatenate([q2, qe2, z9], axis=1)
    kv2t_ref[...] = jnp.concatenate([k2, z2, v2, z2], axis=1)
    sk2_ref[...] = (jnp.dot(h1, sk2w_ref[...], preferred_element_type=jnp.float32)
                    + sk2b_ref[...])


_combine1 = pl.pallas_call(
    _combine1_body,
    out_shape=(
        jax.ShapeDtypeStruct((NPAD, 16), jnp.float32),
        jax.ShapeDtypeStruct((NPAD, 16), jnp.float32),
        jax.ShapeDtypeStruct((NPAD, C), jnp.float32),
    ),
)


def _final_body(p_ref, sk2_ref, e2w_ref, ln2w_ref, ln2b_ref, bt_ref,
                m1w_ref, m1b_ref, m2w_ref, m2b_ref, out_ref):
    p = p_ref[0, :, :] + p_ref[1, :, :]
    w = p[:, C:C + 1]
    u = p[:, C + 1:C + 2]
    conv = (p[:, :C] + u * e2w_ref[...]) / (w + 1e-16) + sk2_ref[...]
    h2 = _layernorm(_elu(conv), ln2w_ref[...], ln2b_ref[...])

    rows = lax.broadcasted_iota(jnp.int32, (NPAD, 1), 0)
    valid = rows < N
    bt = bt_ref[...]
    seg = lax.broadcasted_iota(jnp.int32, (1, B), 1)
    oh = jnp.logical_and(bt == seg, valid)
    ohf = oh.astype(jnp.float32)
    counts = jnp.sum(ohf, axis=0)
    sums = lax.dot_general(ohf, h2, (((0,), (0,)), ((), ())),
                           preferred_element_type=jnp.float32)
    mean_p = sums / jnp.maximum(counts, 1.0)[:, None]

    neg = jnp.float32(-jnp.inf)
    mx_rows = []
    for b in range(B):
        hm = jnp.where(oh[:, b:b + 1], h2, neg)
        mx_rows.append(jnp.max(hm, axis=0, keepdims=True))
    max_p = jnp.concatenate(mx_rows, axis=0)

    pooled = jnp.concatenate([mean_p, max_p], axis=1)
    z = jnp.maximum(
        jnp.dot(pooled, m1w_ref[...], preferred_element_type=jnp.float32)
        + m1b_ref[...], 0.0)
    out_ref[...] = (jnp.dot(z, m2w_ref[...], preferred_element_type=jnp.float32)
                    + m2b_ref[...])


_final = pl.pallas_call(
    _final_body,
    out_shape=jax.ShapeDtypeStruct((B, 1), jnp.float32),
)


def kernel(x, edge_index, edge_attr, batch,
           q1w, q1b, k1w, k1b, v1w, v1b, e1w, sk1w, sk1b,
           q2w, q2b, k2w, k2b, v2w, v2b, e2w, sk2w, sk2b,
           ln1w, ln1b, ln2w, ln2b, m1w, m1b, m2w, m2b):
    f32 = jnp.float32
    xpad = jnp.pad(x, ((0, NPAD - N), (0, 0)))
    src = jnp.pad(edge_index[0], (0, EPAD - E), constant_values=N)
    dst = jnp.pad(edge_index[1], (0, EPAD - E), constant_values=N)
    ea = jnp.pad(edge_attr[:, 0], (0, EPAD - E))
    src3 = src.reshape(NW, KC, CHUNK)
    dst3 = dst.reshape(NW, KC, CHUNK)
    ea3 = ea.reshape(NW, KC, CHUNK)
    z32 = jnp.zeros((NPAD, 32), f32)
    z8 = jnp.zeros((NPAD, 8), f32)

    r = lambda a: a.reshape(1, -1)
    qt, kvt, sk1 = _prep1(xpad, q1w, r(q1b), k1w, r(k1b), v1w, r(v1b),
                          e1w, sk1w, r(sk1b))
    part1 = _edge_l1(qt, kvt, src3, dst3, ea3, z32)
    q2t, kv2t, sk2 = _combine1(part1, sk1, e1w, r(ln1w), r(ln1b),
                               q2w, r(q2b), k2w, r(k2b), v2w, r(v2b),
                               e2w, sk2w, r(sk2b))
    part2 = _edge_l2(q2t, kv2t, src3, dst3, ea3, z8)
    bt = jnp.pad(batch, (0, NPAD - N)).reshape(-1, 1).astype(jnp.int32)
    out = _final(part2, sk2, e2w, r(ln2w), r(ln2b), bt,
                 m1w, r(m1b), m2w, r(m2b))
    return out[:, 0]


# speedup vs baseline: 36.7055x; 36.7055x over previous
import math

import jax
import jax.numpy as jnp
from jax import lax
from jax.experimental import pallas as pl
from jax.experimental.pallas import tpu as pltpu
from jax.experimental.pallas import tpu_sc as plsc

N = 10000
E = 320000
D = 128
H1 = 4
C = 6
B = 64

NC = 2
NS = 16
NW = NC * NS
LANES = 16

NPAD = 10112
ROWS_PT = NPAD // NS
CHUNK = 128
KC = -(-E // (NW * CHUNK))
EPAD = NW * KC * CHUNK

_ISQRT_C = 1.0 / math.sqrt(float(C))


def _make_edge_kernel(heads, ch, qw_, kw_, aw_):
    hc = heads * ch
    qe_off = hc
    v_off = kw_ // 2
    w_off = hc
    u_off = hc + heads
    mesh = plsc.VectorSubcoreMesh(core_axis_name="c", subcore_axis_name="s")

    def body(qt, kvt, src3, dst3, ea3, zr, out,
             idx_s, idx_d, ea_v, q_rows, kv_rows, out_rows, acc):
        cid = lax.axis_index("c")
        sid = lax.axis_index("s")
        wid = cid * NS + sid

        pltpu.sync_copy(zr.at[pl.ds(sid * ROWS_PT, ROWS_PT)],
                        acc.at[pl.ds(sid * ROWS_PT, ROWS_PT)])
        plsc.subcore_barrier()

        def chunk_body(j, carry):
            pltpu.sync_copy(src3.at[wid, j], idx_s)
            pltpu.sync_copy(dst3.at[wid, j], idx_d)
            pltpu.sync_copy(ea3.at[wid, j], ea_v)
            pltpu.sync_copy(kvt.at[idx_s], kv_rows)
            pltpu.sync_copy(qt.at[idx_d], q_rows)
            for g in range(CHUNK // LANES):
                lanes = lax.iota(jnp.int32, 16) + (g * LANES)
                ea_g = ea_v[pl.ds(g * LANES, LANES)]
                for h in range(heads):
                    qe = plsc.load_gather(
                        q_rows, [lanes, jnp.full((16,), qe_off + h, jnp.int32)])
                    alpha = ea_g * qe
                    for c in range(ch):
                        col = jnp.full((16,), h * ch + c, jnp.int32)
                        qv = plsc.load_gather(q_rows, [lanes, col])
                        kv = plsc.load_gather(kv_rows, [lanes, col])
                        alpha = alpha + qv * kv
                    w = jnp.exp(alpha)
                    plsc.store_scatter(
                        out_rows, [lanes, jnp.full((16,), w_off + h, jnp.int32)], w)
                    plsc.store_scatter(
                        out_rows, [lanes, jnp.full((16,), u_off + h, jnp.int32)],
                        w * ea_g)
                    for c in range(ch):
                        vv = plsc.load_gather(
                            kv_rows,
                            [lanes, jnp.full((16,), v_off + h * ch + c, jnp.int32)])
                        plsc.store_scatter(
                            out_rows,
                            [lanes, jnp.full((16,), h * ch + c, jnp.int32)],
                            w * vv)
            pltpu.sync_copy(out_rows, acc.at[idx_d], add=True)
            return carry

        lax.fori_loop(0, KC, chunk_body, 0)
        plsc.subcore_barrier()
        pltpu.sync_copy(acc.at[pl.ds(sid * ROWS_PT, ROWS_PT)],
                        out.at[cid, pl.ds(sid * ROWS_PT, ROWS_PT)])

    return pl.kernel(
        body,
        out_type=jax.ShapeDtypeStruct((NC, NPAD, aw_), jnp.float32),
        mesh=mesh,
        compiler_params=pltpu.CompilerParams(
            needs_layout_passes=False, use_tc_tiling_on_sc=False),
        scratch_types=[
            pltpu.VMEM((CHUNK,), jnp.int32),
            pltpu.VMEM((CHUNK,), jnp.int32),
            pltpu.VMEM((CHUNK,), jnp.float32),
            pltpu.VMEM((CHUNK, qw_), jnp.float32),
            pltpu.VMEM((CHUNK, kw_), jnp.float32),
            pltpu.VMEM((CHUNK, aw_), jnp.float32),
            pltpu.VMEM_SHARED((NPAD, aw_), jnp.float32),
        ],
    )


_edge_l1 = _make_edge_kernel(H1, C, 32, 64, 32)
_edge_l2 = _make_edge_kernel(1, C, 16, 16, 8)


def _elu(x):
    return jnp.where(x > 0, x, jnp.exp(jnp.minimum(x, 0.0)) - 1.0)


def _layernorm(x, w, b):
    mu = jnp.mean(x, axis=-1, keepdims=True)
    var = jnp.mean((x - mu) * (x - mu), axis=-1, keepdims=True)
    return (x - mu) * lax.rsqrt(var + 1e-5) * w + b


def _prep1_body(x_ref, qw_ref, qb_ref, kw_ref, kb_ref, vw_ref, vb_ref,
                e1w_ref, skw_ref, skb_ref, qt_ref, kvt_ref, sk_ref):
    x = x_ref[...]
    q = (jnp.dot(x, qw_ref[...], preferred_element_type=jnp.float32)
         + qb_ref[...]) * _ISQRT_C
    k = jnp.dot(x, kw_ref[...], preferred_element_type=jnp.float32) + kb_ref[...]
    v = jnp.dot(x, vw_ref[...], preferred_element_type=jnp.float32) + vb_ref[...]
    e1 = e1w_ref[...]
    sel = jnp.equal(
        lax.broadcasted_iota(jnp.int32, (H1 * C, H1), 0) // C,
        lax.broadcasted_iota(jnp.int32, (H1 * C, H1), 1)).astype(jnp.float32)
    qe4 = jnp.dot(q * e1, sel, preferred_element_type=jnp.float32)
    z4 = jnp.zeros((NPAD, 4), jnp.float32)
    z8 = jnp.zeros((NPAD, 8), jnp.float32)
    qt_ref[...] = jnp.concatenate([q, qe4, z4], axis=1)
    kvt_ref[...] = jnp.concatenate([k, z8, v, z8], axis=1)
    sk_ref[...] = (jnp.dot(x, skw_ref[...], preferred_element_type=jnp.float32)
                   + skb_ref[...])


_prep1 = pl.pallas_call(
    _prep1_body,
    out_shape=(
        jax.ShapeDtypeStruct((NPAD, 32), jnp.float32),
        jax.ShapeDtypeStruct((NPAD, 64), jnp.float32),
        jax.ShapeDtypeStruct((NPAD, H1 * C), jnp.float32),
    ),
)


def _combine1_body(p_ref, sk_ref, e1w_ref, ln1w_ref, ln1b_ref,
                   q2w_ref, q2b_ref, k2w_ref, k2b_ref, v2w_ref, v2b_ref,
                   e2w_ref, sk2w_ref, sk2b_ref,
                   q2t_ref, kv2t_ref, sk2_ref):
    p = p_ref[0, :, :] + p_ref[1, :, :]
    e1 = e1w_ref[...]
    selT = jnp.equal(
        lax.broadcasted_iota(jnp.int32, (H1, H1 * C), 0),
        lax.broadcasted_iota(jnp.int32, (H1, H1 * C), 1) // C).astype(jnp.float32)
    w24 = jnp.dot(p[:, 24:28], selT, preferred_element_type=jnp.float32)
    u24 = jnp.dot(p[:, 28:32], selT, preferred_element_type=jnp.float32)
    conv = (p[:, 0:24] + u24 * e1) / (w24 + 1e-16) + sk_ref[...]
    h1 = _layernorm(_elu(conv), ln1w_ref[...], ln1b_ref[...])

    q2 = (jnp.dot(h1, q2w_ref[...], preferred_element_type=jnp.float32)
          + q2b_ref[...]) * _ISQRT_C
    k2 = jnp.dot(h1, k2w_ref[...], preferred_element_type=jnp.float32) + k2b_ref[...]
    v2 = jnp.dot(h1, v2w_ref[...], preferred_element_type=jnp.float32) + v2b_ref[...]
    qe2 = jnp.sum(q2 * e2w_ref[...], axis=1, keepdims=True)
    z9 = jnp.zeros((NPAD, 9), jnp.float32)
    z2 = jnp.zeros((NPAD, 2), jnp.float32)
    q2t_ref[...] = jnp.concatenate([q2, qe2, z9], axis=1)
    kv2t_ref[...] = jnp.concatenate([k2, z2, v2, z2], axis=1)
    sk2_ref[...] = (jnp.dot(h1, sk2w_ref[...], preferred_element_type=jnp.float32)
                    + sk2b_ref[...])


_combine1 = pl.pallas_call(
    _combine1_body,
    out_shape=(
        jax.ShapeDtypeStruct((NPAD, 16), jnp.float32),
        jax.ShapeDtypeStruct((NPAD, 16), jnp.float32),
        jax.ShapeDtypeStruct((NPAD, C), jnp.float32),
    ),
)


def _final_body(p_ref, sk2_ref, e2w_ref, ln2w_ref, ln2b_ref, bt_ref,
                m1w_ref, m1b_ref, m2w_ref, m2b_ref, out_ref):
    p = p_ref[0, :, :] + p_ref[1, :, :]
    w = p[:, C:C + 1]
    u = p[:, C + 1:C + 2]
    conv = (p[:, :C] + u * e2w_ref[...]) / (w + 1e-16) + sk2_ref[...]
    h2 = _layernorm(_elu(conv), ln2w_ref[...], ln2b_ref[...])

    rows = lax.broadcasted_iota(jnp.int32, (NPAD, 1), 0)
    valid = rows < N
    bt = bt_ref[...]
    seg = lax.broadcasted_iota(jnp.int32, (1, B), 1)
    oh = jnp.logical_and(bt == seg, valid)
    ohf = oh.astype(jnp.float32)
    counts = jnp.sum(ohf, axis=0)
    sums = lax.dot_general(ohf, h2, (((0,), (0,)), ((), ())),
                           preferred_element_type=jnp.float32)
    mean_p = sums / jnp.maximum(counts, 1.0)[:, None]

    neg = jnp.float32(-jnp.inf)
    mx_rows = []
    for c in range(C):
        hm = jnp.where(oh, h2[:, c:c + 1], neg)
        mx_rows.append(jnp.max(hm, axis=0, keepdims=True))
    max_pT = jnp.concatenate(mx_rows, axis=0)

    m1w = m1w_ref[...]
    z = jnp.maximum(
        jnp.dot(mean_p, m1w[0:C, :], preferred_element_type=jnp.float32)
        + lax.dot_general(max_pT, m1w[C:2 * C, :], (((0,), (0,)), ((), ())),
                          preferred_element_type=jnp.float32)
        + m1b_ref[...], 0.0)
    out_ref[...] = (jnp.dot(z, m2w_ref[...], preferred_element_type=jnp.float32)
                    + m2b_ref[...])


_final = pl.pallas_call(
    _final_body,
    out_shape=jax.ShapeDtypeStruct((B, 1), jnp.float32),
)


def kernel(x, edge_index, edge_attr, batch,
           q1w, q1b, k1w, k1b, v1w, v1b, e1w, sk1w, sk1b,
           q2w, q2b, k2w, k2b, v2w, v2b, e2w, sk2w, sk2b,
           ln1w, ln1b, ln2w, ln2b, m1w, m1b, m2w, m2b):
    f32 = jnp.float32
    xpad = jnp.pad(x, ((0, NPAD - N), (0, 0)))
    pad_idx = N + jnp.arange(EPAD - E, dtype=jnp.int32) % (NPAD - N)
    src = jnp.concatenate([edge_index[0], pad_idx])
    dst = jnp.concatenate([edge_index[1], pad_idx])
    ea = jnp.pad(edge_attr[:, 0], (0, EPAD - E))
    src3 = src.reshape(NW, KC, CHUNK)
    dst3 = dst.reshape(NW, KC, CHUNK)
    ea3 = ea.reshape(NW, KC, CHUNK)
    z32 = jnp.zeros((NPAD, 32), f32)
    z8 = jnp.zeros((NPAD, 8), f32)

    r = lambda a: a.reshape(1, -1)
    qt, kvt, sk1 = _prep1(xpad, q1w, r(q1b), k1w, r(k1b), v1w, r(v1b),
                          e1w, sk1w, r(sk1b))
    part1 = _edge_l1(qt, kvt, src3, dst3, ea3, z32)
    q2t, kv2t, sk2 = _combine1(part1, sk1, e1w, r(ln1w), r(ln1b),
                               q2w, r(q2b), k2w, r(k2b), v2w, r(v2b),
                               e2w, sk2w, r(sk2b))
    part2 = _edge_l2(q2t, kv2t, src3, dst3, ea3, z8)
    bt = jnp.pad(batch, (0, NPAD - N)).reshape(-1, 1).astype(jnp.int32)
    out = _final(part2, sk2, e2w, r(ln2w), r(ln2b), bt,
                 m1w, r(m1b), m2w, r(m2b))
    return out[:, 0]
